# baseline (speedup 1.0000x reference)
import functools

import jax
import jax.numpy as jnp
from jax.experimental import pallas as pl
from jax.experimental.pallas import tpu as pltpu
from jax.experimental.pallas import tpu_sc as plsc

_B, _L, _D = 4, 128, 768
_N = _B * _L * _L
_CHUNK = 2048
_NCHUNK = _N // _CHUNK
_NF = _N // _B
_QUARTER = _NF // 4
_NCHK = _QUARTER // 16


def _phase1_body(w8_ref, bias_ref, x_ref, ys_ref, ye_ref,
                 zs_ref, ze_ref, ls_ref, le_ref, accs, acce):
    i = pl.program_id(0)
    x = x_ref[...]
    wcat = w8_ref[...]
    x_hi = x.astype(jnp.bfloat16)
    dn = (((1,), (0,)), ((), ()))
    res = jax.lax.dot_general(
        x_hi, wcat, dn,
        preferred_element_type=jnp.float32)
    rt = res.T
    zs = rt[0:1, :] + bias_ref[0]
    ze = rt[1:2, :] + bias_ref[1]
    ys_i = ys_ref[0]
    ye_i = ye_ref[0]
    ys = ys_i.astype(jnp.float32)
    ye = ye_i.astype(jnp.float32)
    w = (ys_i >= 0).astype(jnp.float32)
    bce_s = jnp.maximum(zs, 0.0) - zs * ys + jnp.log1p(jnp.exp(-jnp.abs(zs)))
    bce_e = jnp.maximum(ze, 0.0) - ze * ye + jnp.log1p(jnp.exp(-jnp.abs(ze)))

    @pl.when(i == 0)
    def _():
        accs[...] = jnp.zeros_like(accs)
        acce[...] = jnp.zeros_like(acce)

    accs[...] += w * bce_s
    acce[...] += w * bce_e
    neg = jnp.float32(-jnp.inf)
    zs_ref[...] = jnp.where(w > 0, zs, neg)[None]
    ze_ref[...] = jnp.where(w > 0, ze, neg)[None]

    @pl.when(i == _NCHUNK - 1)
    def _():
        ls_ref[0, 0] = jnp.sum(accs[...]) * (1.0 / _N)
        le_ref[0, 0] = jnp.sum(acce[...]) * (1.0 / _N)


_sc_mesh = plsc.VectorSubcoreMesh(core_axis_name="c", subcore_axis_name="s")


@functools.partial(
    pl.kernel,
    mesh=_sc_mesh,
    out_type=jax.ShapeDtypeStruct((2 * _N,), jnp.float32),
    scratch_types=[
        pltpu.VMEM((_QUARTER,), jnp.float32),
        pltpu.VMEM((_QUARTER,), jnp.int32),
        pltpu.VMEM((_L,), jnp.int32),
        pltpu.SMEM((40,), jnp.int32),
    ],
)
def _sc_topk(z_hbm, am_hbm, out_hbm, zv, kv, amv, cnt):
    c = jax.lax.axis_index("c")
    s = jax.lax.axis_index("s")
    b = s // 4
    q = s % 4
    leader = b * 4

    def zslot(j, carry):
        cnt[j] = 0
        return carry

    jax.lax.fori_loop(0, 40, zslot, 0)

    pltpu.sync_copy(am_hbm.at[pl.ds(b * _L, _L)], amv)

    def _lanesum(v):
        t = v[0]
        for j in range(1, 16):
            t = t + v[j]
        return t

    def amsum(j, acc):
        return acc + amv[pl.ds(j * 16, 16)]

    av = jax.lax.fori_loop(0, _L // 16, amsum, jnp.zeros((16,), jnp.int32))
    mask_len = _lanesum(av) - 2
    t = mask_len.astype(jnp.float32) * 0.3
    ti = t.astype(jnp.int32)
    ti = jnp.where(ti.astype(jnp.float32) > t, ti - 1, ti)
    k = jnp.maximum(ti, 5)
    k = jnp.minimum(k, mask_len * mask_len)

    base = (c * _B + b) * _NF + q * _QUARTER
    pltpu.sync_copy(z_hbm.at[pl.ds(base, _QUARTER)], kv)

    def mkkeys(j, carry):
        zb = kv[pl.ds(j * 16, 16)]
        kv[pl.ds(j * 16, 16)] = jnp.where(zb >= 0, zb,
                                          zb ^ jnp.int32(0x7FFFFFFF))
        return carry

    jax.lax.fori_loop(0, _NCHK, mkkeys, 0)
    plsc.subcore_barrier()

    def it(j, lohi):
        lo, hi = lohi
        mid = lo + jax.lax.shift_right_logical(hi - lo, 1)

        def cchunk(t, acc):
            key = kv[pl.ds(t * 16, 16)]
            return acc + jnp.where(key >= mid, 1, 0)

        av = jax.lax.fori_loop(0, _NCHK, cchunk, jnp.zeros((16,), jnp.int32))
        plsc.fetch_and_add(cnt.at[j], _lanesum(av), subcore_id=leader)
        plsc.subcore_barrier()
        total = plsc.fetch_and_add(cnt.at[j], 0, subcore_id=leader)
        big = total >= k
        return jnp.where(big, mid, lo), jnp.where(big, hi, mid)

    lo, _ = jax.lax.fori_loop(
        0, 32, it, (jnp.int32(-(2 ** 31)), jnp.int32(2 ** 31 - 1)))

    def mchunk(t, carry):
        key = kv[pl.ds(t * 16, 16)]
        zv[pl.ds(t * 16, 16)] = jnp.where(key >= lo, 1.0, 0.0)
        return carry

    jax.lax.fori_loop(0, _NCHK, mchunk, 0)
    pltpu.sync_copy(zv, out_hbm.at[pl.ds(base, _QUARTER)])


def kernel(table, attention_mask, table_labels_S, table_labels_E, W_S, b_S, W_E, b_E):
    x = table.reshape(_N, _D)
    ys3 = table_labels_S.reshape(_NCHUNK, 1, _CHUNK)
    ye3 = table_labels_E.reshape(_NCHUNK, 1, _CHUNK)
    w2 = jnp.zeros((_D, 8), jnp.float32)
    w2 = w2.at[:, 0].set(W_S[:, 0]).at[:, 1].set(W_E[:, 0])
    w8 = w2.astype(jnp.bfloat16)
    bias = jnp.concatenate([b_S, b_E])

    zs, ze, ls, le = pl.pallas_call(
        _phase1_body,
        grid=(_NCHUNK,),
        in_specs=[
            pl.BlockSpec((_D, 8), lambda i: (0, 0)),
            pl.BlockSpec(memory_space=pltpu.SMEM),
            pl.BlockSpec((_CHUNK, _D), lambda i: (i, 0)),
            pl.BlockSpec((1, 1, _CHUNK), lambda i: (i, 0, 0)),
            pl.BlockSpec((1, 1, _CHUNK), lambda i: (i, 0, 0)),
        ],
        out_specs=[
            pl.BlockSpec((1, 1, _CHUNK), lambda i: (i, 0, 0)),
            pl.BlockSpec((1, 1, _CHUNK), lambda i: (i, 0, 0)),
            pl.BlockSpec((1, 1), lambda i: (0, 0), memory_space=pltpu.SMEM),
            pl.BlockSpec((1, 1), lambda i: (0, 0), memory_space=pltpu.SMEM),
        ],
        out_shape=[
            jax.ShapeDtypeStruct((_NCHUNK, 1, _CHUNK), jnp.float32),
            jax.ShapeDtypeStruct((_NCHUNK, 1, _CHUNK), jnp.float32),
            jax.ShapeDtypeStruct((1, 1), jnp.float32),
            jax.ShapeDtypeStruct((1, 1), jnp.float32),
        ],
        scratch_shapes=[
            pltpu.VMEM((1, _CHUNK), jnp.float32),
            pltpu.VMEM((1, _CHUNK), jnp.float32),
        ],
    )(w8, bias, x, ys3, ye3)

    z_flat = jnp.concatenate([zs.reshape(_N), ze.reshape(_N)])
    z_bits = jax.lax.bitcast_convert_type(z_flat, jnp.int32)
    masks = _sc_topk(z_bits, attention_mask.reshape(_B * _L))
    ms = masks[:_N].reshape(_B, _L, _L)
    me = masks[_N:].reshape(_B, _L, _L)
    return (ls[0, 0], le[0, 0], ms.astype(bool), me.astype(bool))


# speedup vs baseline: 1.2042x; 1.2042x over previous
import functools

import jax
import jax.numpy as jnp
from jax.experimental import pallas as pl
from jax.experimental.pallas import tpu as pltpu
from jax.experimental.pallas import tpu_sc as plsc

_B, _L, _D = 4, 128, 768
_N = _B * _L * _L
_CHUNK = 2048
_NCHUNK = _N // _CHUNK
_NF = _N // _B
_QUARTER = _NF // 4
_NCHK = _QUARTER // 16


def _phase1_body(w8_ref, bias_ref, x_ref, ys_ref, ye_ref,
                 zs_ref, ze_ref, ls_ref, le_ref, accs, acce):
    i = pl.program_id(0)
    x = x_ref[...]
    wcat = w8_ref[...]
    x_hi = x.astype(jnp.bfloat16)
    dn = (((1,), (0,)), ((), ()))
    res = jax.lax.dot_general(
        x_hi, wcat, dn,
        preferred_element_type=jnp.float32)
    rt = res.T
    zs = rt[0:1, :] + bias_ref[0]
    ze = rt[1:2, :] + bias_ref[1]
    ys_i = ys_ref[0]
    ye_i = ye_ref[0]
    ys = ys_i.astype(jnp.float32)
    ye = ye_i.astype(jnp.float32)
    w = (ys_i >= 0).astype(jnp.float32)
    bce_s = jnp.maximum(zs, 0.0) - zs * ys + jnp.log1p(jnp.exp(-jnp.abs(zs)))
    bce_e = jnp.maximum(ze, 0.0) - ze * ye + jnp.log1p(jnp.exp(-jnp.abs(ze)))

    @pl.when(i == 0)
    def _():
        accs[...] = jnp.zeros_like(accs)
        acce[...] = jnp.zeros_like(acce)

    accs[...] += w * bce_s
    acce[...] += w * bce_e
    neg = jnp.float32(-jnp.inf)
    zs_ref[...] = jnp.where(w > 0, zs, neg)[None]
    ze_ref[...] = jnp.where(w > 0, ze, neg)[None]

    @pl.when(i == _NCHUNK - 1)
    def _():
        ls_ref[0, 0] = jnp.sum(accs[...]) * (1.0 / _N)
        le_ref[0, 0] = jnp.sum(acce[...]) * (1.0 / _N)


_sc_mesh = plsc.VectorSubcoreMesh(core_axis_name="c", subcore_axis_name="s")


@functools.partial(
    pl.kernel,
    mesh=_sc_mesh,
    out_type=jax.ShapeDtypeStruct((2 * _N,), jnp.float32),
    scratch_types=[
        pltpu.VMEM((_QUARTER,), jnp.float32),
        pltpu.VMEM((_QUARTER,), jnp.int32),
        pltpu.VMEM((_L,), jnp.int32),
        pltpu.SMEM((40,), jnp.int32),
    ],
)
def _sc_topk(z_hbm, am_hbm, out_hbm, zv, kv, amv, cnt):
    c = jax.lax.axis_index("c")
    s = jax.lax.axis_index("s")
    b = s // 4
    q = s % 4
    leader = b * 4

    def zslot(j, carry):
        cnt[j] = 0
        return carry

    jax.lax.fori_loop(0, 40, zslot, 0)

    pltpu.sync_copy(am_hbm.at[pl.ds(b * _L, _L)], amv)

    def _lanesum(v):
        t = v[0]
        for j in range(1, 16):
            t = t + v[j]
        return t

    def amsum(j, acc):
        return acc + amv[pl.ds(j * 16, 16)]

    av = jax.lax.fori_loop(0, _L // 16, amsum, jnp.zeros((16,), jnp.int32))
    mask_len = _lanesum(av) - 2
    t = mask_len.astype(jnp.float32) * 0.3
    ti = t.astype(jnp.int32)
    ti = jnp.where(ti.astype(jnp.float32) > t, ti - 1, ti)
    k = jnp.maximum(ti, 5)
    k = jnp.minimum(k, mask_len * mask_len)

    base = (c * _B + b) * _NF + q * _QUARTER
    pltpu.sync_copy(z_hbm.at[pl.ds(base, _QUARTER)], kv)

    def mkkeys(j, carry):
        zb = kv[pl.ds(j * 16, 16)]
        kv[pl.ds(j * 16, 16)] = jnp.where(zb >= 0, zb,
                                          zb ^ jnp.int32(0x7FFFFFFF))
        return carry

    jax.lax.fori_loop(0, _NCHK, mkkeys, 0)
    plsc.subcore_barrier()

    def it(j, lohi):
        lo, hi = lohi
        mid = lo + jax.lax.shift_right_logical(hi - lo, 1)

        av = jnp.zeros((16,), jnp.int32)
        for t in range(_NCHK):
            key = kv[pl.ds(t * 16, 16)]
            av = av + jnp.where(key >= mid, 1, 0)
        plsc.fetch_and_add(cnt.at[j], _lanesum(av), subcore_id=leader)
        plsc.subcore_barrier()
        total = plsc.fetch_and_add(cnt.at[j], 0, subcore_id=leader)
        big = total >= k
        return jnp.where(big, mid, lo), jnp.where(big, hi, mid)

    lo, _ = jax.lax.fori_loop(
        0, 32, it, (jnp.int32(-(2 ** 31)), jnp.int32(2 ** 31 - 1)))

    def mchunk(t, carry):
        key = kv[pl.ds(t * 16, 16)]
        zv[pl.ds(t * 16, 16)] = jnp.where(key >= lo, 1.0, 0.0)
        return carry

    jax.lax.fori_loop(0, _NCHK, mchunk, 0)
    pltpu.sync_copy(zv, out_hbm.at[pl.ds(base, _QUARTER)])


def kernel(table, attention_mask, table_labels_S, table_labels_E, W_S, b_S, W_E, b_E):
    x = table.reshape(_N, _D)
    ys3 = table_labels_S.reshape(_NCHUNK, 1, _CHUNK)
    ye3 = table_labels_E.reshape(_NCHUNK, 1, _CHUNK)
    w2 = jnp.zeros((_D, 8), jnp.float32)
    w2 = w2.at[:, 0].set(W_S[:, 0]).at[:, 1].set(W_E[:, 0])
    w8 = w2.astype(jnp.bfloat16)
    bias = jnp.concatenate([b_S, b_E])

    zs, ze, ls, le = pl.pallas_call(
        _phase1_body,
        grid=(_NCHUNK,),
        in_specs=[
            pl.BlockSpec((_D, 8), lambda i: (0, 0)),
            pl.BlockSpec(memory_space=pltpu.SMEM),
            pl.BlockSpec((_CHUNK, _D), lambda i: (i, 0)),
            pl.BlockSpec((1, 1, _CHUNK), lambda i: (i, 0, 0)),
            pl.BlockSpec((1, 1, _CHUNK), lambda i: (i, 0, 0)),
        ],
        out_specs=[
            pl.BlockSpec((1, 1, _CHUNK), lambda i: (i, 0, 0)),
            pl.BlockSpec((1, 1, _CHUNK), lambda i: (i, 0, 0)),
            pl.BlockSpec((1, 1), lambda i: (0, 0), memory_space=pltpu.SMEM),
            pl.BlockSpec((1, 1), lambda i: (0, 0), memory_space=pltpu.SMEM),
        ],
        out_shape=[
            jax.ShapeDtypeStruct((_NCHUNK, 1, _CHUNK), jnp.float32),
            jax.ShapeDtypeStruct((_NCHUNK, 1, _CHUNK), jnp.float32),
            jax.ShapeDtypeStruct((1, 1), jnp.float32),
            jax.ShapeDtypeStruct((1, 1), jnp.float32),
        ],
        scratch_shapes=[
            pltpu.VMEM((1, _CHUNK), jnp.float32),
            pltpu.VMEM((1, _CHUNK), jnp.float32),
        ],
    )(w8, bias, x, ys3, ye3)

    z_flat = jnp.concatenate([zs.reshape(_N), ze.reshape(_N)])
    z_bits = jax.lax.bitcast_convert_type(z_flat, jnp.int32)
    masks = _sc_topk(z_bits, attention_mask.reshape(_B * _L))
    ms = masks[:_N].reshape(_B, _L, _L)
    me = masks[_N:].reshape(_B, _L, _L)
    return (ls[0, 0], le[0, 0], ms.astype(bool), me.astype(bool))


# speedup vs baseline: 1.2160x; 1.0098x over previous
import functools

import jax
import jax.numpy as jnp
from jax.experimental import pallas as pl
from jax.experimental.pallas import tpu as pltpu
from jax.experimental.pallas import tpu_sc as plsc

_B, _L, _D = 4, 128, 768
_N = _B * _L * _L
_CHUNK = 2048
_NCHUNK = _N // _CHUNK
_NF = _N // _B
_QUARTER = _NF // 4
_NCHK = _QUARTER // 16


def _phase1_body(w8_ref, bias_ref, x_ref, ys_ref, ye_ref,
                 zs_ref, ze_ref, ls_ref, le_ref, accs, acce):
    i = pl.program_id(0)
    x = x_ref[...]
    wcat = w8_ref[...]
    x_hi = x.astype(jnp.bfloat16)
    dn = (((1,), (0,)), ((), ()))
    res = jax.lax.dot_general(
        x_hi, wcat, dn,
        preferred_element_type=jnp.float32)
    rt = res.T
    zs = rt[0:1, :] + bias_ref[0]
    ze = rt[1:2, :] + bias_ref[1]
    ys_i = ys_ref[0]
    ye_i = ye_ref[0]
    ys = ys_i.astype(jnp.float32)
    ye = ye_i.astype(jnp.float32)
    w = (ys_i >= 0).astype(jnp.float32)
    bce_s = jnp.maximum(zs, 0.0) - zs * ys + jnp.log1p(jnp.exp(-jnp.abs(zs)))
    bce_e = jnp.maximum(ze, 0.0) - ze * ye + jnp.log1p(jnp.exp(-jnp.abs(ze)))

    @pl.when(i == 0)
    def _():
        accs[...] = jnp.zeros_like(accs)
        acce[...] = jnp.zeros_like(acce)

    accs[...] += w * bce_s
    acce[...] += w * bce_e
    neg = jnp.float32(-jnp.inf)
    zs_ref[...] = jnp.where(w > 0, zs, neg)[None]
    ze_ref[...] = jnp.where(w > 0, ze, neg)[None]

    @pl.when(i == _NCHUNK - 1)
    def _():
        ls_ref[0, 0] = jnp.sum(accs[...]) * (1.0 / _N)
        le_ref[0, 0] = jnp.sum(acce[...]) * (1.0 / _N)


_sc_mesh = plsc.VectorSubcoreMesh(core_axis_name="c", subcore_axis_name="s")


@functools.partial(
    pl.kernel,
    mesh=_sc_mesh,
    out_type=jax.ShapeDtypeStruct((2 * _N,), jnp.float32),
    scratch_types=[
        pltpu.VMEM((_QUARTER,), jnp.float32),
        pltpu.VMEM((_QUARTER,), jnp.int32),
        pltpu.VMEM((_L,), jnp.int32),
        pltpu.SMEM((40,), jnp.int32),
    ],
)
def _sc_topk(z_hbm, am_hbm, out_hbm, zv, kv, amv, cnt):
    c = jax.lax.axis_index("c")
    s = jax.lax.axis_index("s")
    b = s // 4
    q = s % 4
    leader = b * 4

    def zslot(j, carry):
        cnt[j] = 0
        return carry

    jax.lax.fori_loop(0, 40, zslot, 0)

    pltpu.sync_copy(am_hbm.at[pl.ds(b * _L, _L)], amv)

    def _lanesum(v):
        t = v[0]
        for j in range(1, 16):
            t = t + v[j]
        return t

    def amsum(j, acc):
        return acc + amv[pl.ds(j * 16, 16)]

    av = jax.lax.fori_loop(0, _L // 16, amsum, jnp.zeros((16,), jnp.int32))
    mask_len = _lanesum(av) - 2
    t = mask_len.astype(jnp.float32) * 0.3
    ti = t.astype(jnp.int32)
    ti = jnp.where(ti.astype(jnp.float32) > t, ti - 1, ti)
    k = jnp.maximum(ti, 5)
    k = jnp.minimum(k, mask_len * mask_len)

    base = (c * _B + b) * _NF + q * _QUARTER
    pltpu.sync_copy(z_hbm.at[pl.ds(base, _QUARTER)], kv)

    for j in range(_NCHK):
        zb = kv[pl.ds(j * 16, 16)]
        kv[pl.ds(j * 16, 16)] = jnp.where(zb >= 0, zb,
                                          zb ^ jnp.int32(0x7FFFFFFF))
    plsc.subcore_barrier()

    def it(j, lohi):
        lo, hi = lohi
        mid = lo + jax.lax.shift_right_logical(hi - lo, 1)

        av = jnp.zeros((16,), jnp.int32)
        for t in range(_NCHK):
            key = kv[pl.ds(t * 16, 16)]
            av = av + jnp.where(key >= mid, 1, 0)
        plsc.fetch_and_add(cnt.at[j], _lanesum(av), subcore_id=leader)
        plsc.subcore_barrier()
        total = plsc.fetch_and_add(cnt.at[j], 0, subcore_id=leader)
        big = total >= k
        return jnp.where(big, mid, lo), jnp.where(big, hi, mid)

    lo, _ = jax.lax.fori_loop(
        0, 32, it, (jnp.int32(-(2 ** 31)), jnp.int32(2 ** 31 - 1)))

    for t in range(_NCHK):
        key = kv[pl.ds(t * 16, 16)]
        zv[pl.ds(t * 16, 16)] = jnp.where(key >= lo, 1.0, 0.0)
    pltpu.sync_copy(zv, out_hbm.at[pl.ds(base, _QUARTER)])


def kernel(table, attention_mask, table_labels_S, table_labels_E, W_S, b_S, W_E, b_E):
    x = table.reshape(_N, _D)
    ys3 = table_labels_S.reshape(_NCHUNK, 1, _CHUNK)
    ye3 = table_labels_E.reshape(_NCHUNK, 1, _CHUNK)
    w2 = jnp.zeros((_D, 8), jnp.float32)
    w2 = w2.at[:, 0].set(W_S[:, 0]).at[:, 1].set(W_E[:, 0])
    w8 = w2.astype(jnp.bfloat16)
    bias = jnp.concatenate([b_S, b_E])

    zs, ze, ls, le = pl.pallas_call(
        _phase1_body,
        grid=(_NCHUNK,),
        in_specs=[
            pl.BlockSpec((_D, 8), lambda i: (0, 0)),
            pl.BlockSpec(memory_space=pltpu.SMEM),
            pl.BlockSpec((_CHUNK, _D), lambda i: (i, 0)),
            pl.BlockSpec((1, 1, _CHUNK), lambda i: (i, 0, 0)),
            pl.BlockSpec((1, 1, _CHUNK), lambda i: (i, 0, 0)),
        ],
        out_specs=[
            pl.BlockSpec((1, 1, _CHUNK), lambda i: (i, 0, 0)),
            pl.BlockSpec((1, 1, _CHUNK), lambda i: (i, 0, 0)),
            pl.BlockSpec((1, 1), lambda i: (0, 0), memory_space=pltpu.SMEM),
            pl.BlockSpec((1, 1), lambda i: (0, 0), memory_space=pltpu.SMEM),
        ],
        out_shape=[
            jax.ShapeDtypeStruct((_NCHUNK, 1, _CHUNK), jnp.float32),
            jax.ShapeDtypeStruct((_NCHUNK, 1, _CHUNK), jnp.float32),
            jax.ShapeDtypeStruct((1, 1), jnp.float32),
            jax.ShapeDtypeStruct((1, 1), jnp.float32),
        ],
        scratch_shapes=[
            pltpu.VMEM((1, _CHUNK), jnp.float32),
            pltpu.VMEM((1, _CHUNK), jnp.float32),
        ],
    )(w8, bias, x, ys3, ye3)

    z_flat = jnp.concatenate([zs.reshape(_N), ze.reshape(_N)])
    z_bits = jax.lax.bitcast_convert_type(z_flat, jnp.int32)
    masks = _sc_topk(z_bits, attention_mask.reshape(_B * _L))
    ms = masks[:_N].reshape(_B, _L, _L)
    me = masks[_N:].reshape(_B, _L, _L)
    return (ls[0, 0], le[0, 0], ms.astype(bool), me.astype(bool))
